# baseline (speedup 1.0000x reference)
import functools

import jax
import jax.numpy as jnp
from jax.experimental import pallas as pl
from jax.experimental.pallas import tpu as pltpu


def _fused_kernel(adj_ref, x_ref, w1_ref, w2_ref, w3_ref, et_ref, ef_ref,
                  o_ref, xw_ref, z_ref, fold_ref,
                  *, tm, n_enc, n_true_tiles):
    n, h = z_ref.shape
    te = et_ref.shape[0]
    t = pl.program_id(0)

    @pl.when(t == 0)
    def _():
        xb = x_ref[...].astype(jnp.bfloat16)
        wb = w1_ref[...].astype(jnp.bfloat16)
        xw_ref[...] = jnp.dot(xb, wb, preferred_element_type=jnp.float32
                              ).astype(jnp.bfloat16)
        w3r = w3_ref[...]
        vab = jax.lax.dot_general(
            w3r[:, :h], w2_ref[...], (((1,), (1,)), ((), ())),
            preferred_element_type=jnp.float32)
        fold_ref[...] = jnp.concatenate([vab, w3r[:, h:]], axis=1)

    @pl.when(t < n_enc)
    def _():
        ab = adj_ref[...].astype(jnp.bfloat16)
        zb = jnp.dot(ab, xw_ref[...],
                     preferred_element_type=jnp.float32).astype(jnp.bfloat16)
        z_ref[pl.ds(pl.multiple_of(t * tm, tm), tm), :] = zb

    @pl.when(t >= n_enc)
    def _():
        d = t - n_enc
        blk = jnp.where(d < n_true_tiles, et_ref[...], ef_ref[...])
        idx = jnp.concatenate([blk[:, 0:1], blk[:, 1:2]], axis=0)
        node_ids = jax.lax.broadcasted_iota(jnp.int16, (2 * te, n), 1)
        oh = jnp.where(node_ids == idx.astype(jnp.int16),
                       jnp.bfloat16(1), jnp.bfloat16(0))
        g = jnp.dot(oh, z_ref[...], preferred_element_type=jnp.float32)
        zi = g[:te]
        zj = g[te:]
        v2a = fold_ref[:, :h]
        v2b = fold_ref[:, h:2 * h]
        w3b = fold_ref[:, 2 * h:]
        m = (zi * zj * w3b
             + jnp.maximum(zi, 0.0) * v2a
             + jnp.maximum(zj, 0.0) * v2b)
        logits = jnp.sum(m, axis=1, keepdims=True)
        o_ref[...] = 1.0 / (1.0 + jnp.exp(-logits))


def _pick_tile(n, desired):
    for t in (desired, 1024, 512, 256, 128):
        if t <= n and n % t == 0 and t % 128 == 0:
            return t
    return n


def _run(adj, x, w1, w2, w3r, te_arr, fe_arr, *, TE, n_true_tiles, n_tiles):
    f32 = jnp.float32
    N = adj.shape[0]
    Din, H = w1.shape
    tm = _pick_tile(N, 1024)
    n_enc = N // tm
    E_out = n_tiles * TE
    last_enc = n_enc - 1
    last_t = max(n_true_tiles - 1, 0)
    last_f = max(n_tiles - n_true_tiles - 1, 0)
    last_o = n_tiles - 1

    body = functools.partial(_fused_kernel, tm=tm, n_enc=n_enc,
                             n_true_tiles=n_true_tiles)
    return pl.pallas_call(
        body,
        out_shape=jax.ShapeDtypeStruct((E_out, 1), f32),
        grid=(n_enc + n_tiles,),
        in_specs=[
            pl.BlockSpec((tm, N), lambda t: (jnp.minimum(t, last_enc), 0)),
            pl.BlockSpec((N, Din), lambda t: (0, 0)),
            pl.BlockSpec((Din, H), lambda t: (0, 0)),
            pl.BlockSpec((2 * H, H), lambda t: (0, 0)),
            pl.BlockSpec((1, 2 * H), lambda t: (0, 0)),
            pl.BlockSpec((TE, 2),
                         lambda t: (jnp.clip(t - n_enc, 0, last_t), 0)),
            pl.BlockSpec((TE, 2),
                         lambda t: (jnp.clip(t - n_enc - n_true_tiles, 0, last_f), 0)),
        ],
        out_specs=pl.BlockSpec((TE, 1),
                               lambda t: (jnp.clip(t - n_enc, 0, last_o), 0)),
        scratch_shapes=[pltpu.VMEM((N, H), jnp.bfloat16),
                        pltpu.VMEM((N, H), jnp.bfloat16),
                        pltpu.VMEM((1, 3 * H), f32)],
        compiler_params=pltpu.CompilerParams(
            dimension_semantics=("arbitrary",),
            vmem_limit_bytes=57 * 1024 * 1024),
    )(adj, x, w1, w2, w3r, te_arr, fe_arr)


def kernel(x, adj, weight, weight_two, weight_three, train_edges, train_false_edges):
    f32 = jnp.float32
    H = weight.shape[1]
    w2 = jnp.asarray(weight_two, f32)
    w3r = jnp.asarray(weight_three, f32).reshape(1, 2 * H)
    te_arr = jnp.asarray(train_edges, jnp.int32)
    fe_arr = jnp.asarray(train_false_edges, jnp.int32)
    E_true, E_false = te_arr.shape[0], fe_arr.shape[0]
    E = E_true + E_false
    TE = 1024

    if E_true % TE == 0 and E_false % TE == 0:
        out = _run(adj, x, weight, w2, w3r, te_arr, fe_arr,
                   TE=TE, n_true_tiles=E_true // TE, n_tiles=E // TE)
        return out
    edges = jnp.concatenate([te_arr, fe_arr], axis=0)
    n_tiles = int(pl.cdiv(E, TE))
    edges = jnp.pad(edges, ((0, n_tiles * TE - E), (0, 0)))
    out = _run(adj, x, weight, w2, w3r, edges, edges,
               TE=TE, n_true_tiles=n_tiles, n_tiles=n_tiles)
    return out[:E]


# speedup vs baseline: 1.4490x; 1.4490x over previous
import functools

import jax
import jax.numpy as jnp
from jax.experimental import pallas as pl
from jax.experimental.pallas import tpu as pltpu


def _fused_kernel(adj_ref, x_ref, w1_ref, w2_ref, w3_ref, ei_ref, ej_ref,
                  o_ref, xw_ref, z3_ref, fold_ref, ti_ref, tj_ref,
                  *, tm, n_enc):
    h = w1_ref.shape[1]
    te = ti_ref.shape[0]
    t = pl.program_id(0)

    @pl.when(t == 0)
    def _():
        xb = x_ref[...].astype(jnp.bfloat16)
        wb = w1_ref[...].astype(jnp.bfloat16)
        xw_ref[...] = jnp.dot(xb, wb, preferred_element_type=jnp.float32
                              ).astype(jnp.bfloat16)
        w3r = w3_ref[...]
        vab = jax.lax.dot_general(
            w3r[:, :h], w2_ref[...], (((1,), (1,)), ((), ())),
            preferred_element_type=jnp.float32)
        fold_ref[...] = jnp.concatenate(
            [vab[:, :128], vab[:, 128:256], vab[:, 256:384], vab[:, 384:],
             w3r[:, h:h + 128], w3r[:, h + 128:]], axis=0)

    @pl.when(t < n_enc)
    def _():
        ab = adj_ref[...].astype(jnp.bfloat16)
        zb = jnp.dot(ab, xw_ref[...],
                     preferred_element_type=jnp.float32).astype(jnp.bfloat16)
        z3_ref[pl.ds(t * tm, tm)] = zb.astype(jnp.float32).reshape(tm, 2, 128)

    @pl.when(t >= n_enc)
    def _():
        for e in range(te):
            ti_ref[e] = z3_ref[ei_ref[0, 0, e]]
            tj_ref[e] = z3_ref[ej_ref[0, 0, e]]
        zi = ti_ref[...]
        zj = tj_ref[...]
        v2a = fold_ref[0:2]
        v2b = fold_ref[2:4]
        w3b = fold_ref[4:6]
        m = (zi * zj * w3b
             + jnp.maximum(zi, 0.0) * v2a
             + jnp.maximum(zj, 0.0) * v2b)
        logits = jnp.sum(jnp.sum(m, axis=1), axis=1, keepdims=True)
        o_ref[...] = 1.0 / (1.0 + jnp.exp(-logits))


def _pick_tile(n, desired):
    for t in (desired, 512, 256, 128):
        if t <= n and n % t == 0 and t % 128 == 0:
            return t
    return n


def kernel(x, adj, weight, weight_two, weight_three, train_edges, train_false_edges):
    f32 = jnp.float32
    N = adj.shape[0]
    Din, H = weight.shape
    w2 = jnp.asarray(weight_two, f32)
    w3r = jnp.asarray(weight_three, f32).reshape(1, 2 * H)
    edges = jnp.concatenate([jnp.asarray(train_edges, jnp.int32),
                             jnp.asarray(train_false_edges, jnp.int32)], axis=0)
    E = edges.shape[0]
    TE = 512
    n_tiles = int(pl.cdiv(E, TE))
    E_pad = n_tiles * TE
    edges = jnp.pad(edges, ((0, E_pad - E), (0, 0)))
    ei = edges[:, 0].reshape(n_tiles, 1, TE)
    ej = edges[:, 1].reshape(n_tiles, 1, TE)

    tm = _pick_tile(N, 512)
    n_enc = N // tm
    last_enc = n_enc - 1
    last_o = n_tiles - 1

    body = functools.partial(_fused_kernel, tm=tm, n_enc=n_enc)
    out = pl.pallas_call(
        body,
        out_shape=jax.ShapeDtypeStruct((E_pad, 1), f32),
        grid=(n_enc + n_tiles,),
        in_specs=[
            pl.BlockSpec((tm, N), lambda t: (jnp.minimum(t, last_enc), 0)),
            pl.BlockSpec((N, Din), lambda t: (0, 0)),
            pl.BlockSpec((Din, H), lambda t: (0, 0)),
            pl.BlockSpec((2 * H, H), lambda t: (0, 0)),
            pl.BlockSpec((1, 2 * H), lambda t: (0, 0)),
            pl.BlockSpec((1, 1, TE),
                         lambda t: (jnp.clip(t - n_enc, 0, last_o), 0, 0),
                         memory_space=pltpu.SMEM),
            pl.BlockSpec((1, 1, TE),
                         lambda t: (jnp.clip(t - n_enc, 0, last_o), 0, 0),
                         memory_space=pltpu.SMEM),
        ],
        out_specs=pl.BlockSpec((TE, 1),
                               lambda t: (jnp.clip(t - n_enc, 0, last_o), 0)),
        scratch_shapes=[pltpu.VMEM((N, H), jnp.bfloat16),
                        pltpu.VMEM((N, 2, 128), f32),
                        pltpu.VMEM((6, 128), f32),
                        pltpu.VMEM((TE, 2, 128), f32),
                        pltpu.VMEM((TE, 2, 128), f32)],
        compiler_params=pltpu.CompilerParams(
            dimension_semantics=("arbitrary",),
            vmem_limit_bytes=57 * 1024 * 1024),
    )(adj, x, weight, w2, w3r, ei, ej)
    return out[:E]


# speedup vs baseline: 1.4723x; 1.0161x over previous
import functools

import jax
import jax.numpy as jnp
from jax.experimental import pallas as pl
from jax.experimental.pallas import tpu as pltpu


def _fused_kernel(adj_ref, x_ref, w1_ref, w2_ref, w3_ref, ei_ref, ej_ref,
                  o_ref, xw_ref, z3_ref, fold_ref, ti_ref, tj_ref,
                  *, tm, n_enc):
    h = w1_ref.shape[1]
    te = ti_ref.shape[0]
    t = pl.program_id(0)

    @pl.when(t == 0)
    def _():
        xb = x_ref[...].astype(jnp.bfloat16)
        wb = w1_ref[...].astype(jnp.bfloat16)
        xw_ref[...] = jnp.dot(xb, wb, preferred_element_type=jnp.float32
                              ).astype(jnp.bfloat16)
        w3r = w3_ref[...]
        vab = jax.lax.dot_general(
            w3r[:, :h], w2_ref[...], (((1,), (1,)), ((), ())),
            preferred_element_type=jnp.float32)
        fold_ref[...] = jnp.concatenate(
            [vab[:, :128], vab[:, 128:256], vab[:, 256:384], vab[:, 384:],
             w3r[:, h:h + 128], w3r[:, h + 128:]], axis=0)

    @pl.when(t < n_enc)
    def _():
        ab = adj_ref[...].astype(jnp.bfloat16)
        zb = jnp.dot(ab, xw_ref[...],
                     preferred_element_type=jnp.float32).astype(jnp.bfloat16)
        z3_ref[pl.ds(t * tm, tm)] = zb.astype(jnp.float32).reshape(tm, 2, 128)

    @pl.when(t >= n_enc)
    def _():
        for e in range(te):
            ti_ref[e] = z3_ref[ei_ref[0, 0, e]]
            tj_ref[e] = z3_ref[ej_ref[0, 0, e]]
        zi = ti_ref[...]
        zj = tj_ref[...]
        v2a = fold_ref[0:2]
        v2b = fold_ref[2:4]
        w3b = fold_ref[4:6]
        m = (zi * zj * w3b
             + jnp.maximum(zi, 0.0) * v2a
             + jnp.maximum(zj, 0.0) * v2b)
        logits = jnp.sum(jnp.sum(m, axis=1), axis=1, keepdims=True)
        o_ref[...] = 1.0 / (1.0 + jnp.exp(-logits))


def _pick_tile(n, desired):
    for t in (desired, 512, 256, 128):
        if t <= n and n % t == 0 and t % 128 == 0:
            return t
    return n


def kernel(x, adj, weight, weight_two, weight_three, train_edges, train_false_edges):
    f32 = jnp.float32
    N = adj.shape[0]
    Din, H = weight.shape
    w2 = jnp.asarray(weight_two, f32)
    w3r = jnp.asarray(weight_three, f32).reshape(1, 2 * H)
    edges = jnp.concatenate([jnp.asarray(train_edges, jnp.int32),
                             jnp.asarray(train_false_edges, jnp.int32)], axis=0)
    E = edges.shape[0]
    TE = 1024
    n_tiles = int(pl.cdiv(E, TE))
    E_pad = n_tiles * TE
    edges = jnp.pad(edges, ((0, E_pad - E), (0, 0)))
    ei = edges[:, 0].reshape(n_tiles, 1, TE)
    ej = edges[:, 1].reshape(n_tiles, 1, TE)

    tm = _pick_tile(N, 512)
    n_enc = N // tm
    last_enc = n_enc - 1
    last_o = n_tiles - 1

    body = functools.partial(_fused_kernel, tm=tm, n_enc=n_enc)
    out = pl.pallas_call(
        body,
        out_shape=jax.ShapeDtypeStruct((E_pad, 1), f32),
        grid=(n_enc + n_tiles,),
        in_specs=[
            pl.BlockSpec((tm, N), lambda t: (jnp.minimum(t, last_enc), 0)),
            pl.BlockSpec((N, Din), lambda t: (0, 0)),
            pl.BlockSpec((Din, H), lambda t: (0, 0)),
            pl.BlockSpec((2 * H, H), lambda t: (0, 0)),
            pl.BlockSpec((1, 2 * H), lambda t: (0, 0)),
            pl.BlockSpec((1, 1, TE),
                         lambda t: (jnp.clip(t - n_enc, 0, last_o), 0, 0),
                         memory_space=pltpu.SMEM),
            pl.BlockSpec((1, 1, TE),
                         lambda t: (jnp.clip(t - n_enc, 0, last_o), 0, 0),
                         memory_space=pltpu.SMEM),
        ],
        out_specs=pl.BlockSpec((TE, 1),
                               lambda t: (jnp.clip(t - n_enc, 0, last_o), 0)),
        scratch_shapes=[pltpu.VMEM((N, H), jnp.bfloat16),
                        pltpu.VMEM((N, 2, 128), f32),
                        pltpu.VMEM((6, 128), f32),
                        pltpu.VMEM((TE, 2, 128), f32),
                        pltpu.VMEM((TE, 2, 128), f32)],
        compiler_params=pltpu.CompilerParams(
            dimension_semantics=("arbitrary",),
            vmem_limit_bytes=57 * 1024 * 1024),
    )(adj, x, weight, w2, w3r, ei, ej)
    return out[:E]


# speedup vs baseline: 1.7348x; 1.1783x over previous
import functools

import jax
import jax.numpy as jnp
from jax.experimental import pallas as pl
from jax.experimental.pallas import tpu as pltpu


def _fused_kernel(adj_ref, x_ref, w1_ref, w2_ref, w3_ref, ei_ref, ej_ref,
                  o_ref, xw_ref, z3_ref, fold_ref, ti_ref, tj_ref,
                  *, tm, n_enc):
    h = w1_ref.shape[1]
    te = ti_ref.shape[0]
    t = pl.program_id(0)

    @pl.when(t == 0)
    def _():
        xb = x_ref[...].astype(jnp.bfloat16)
        wb = w1_ref[...].astype(jnp.bfloat16)
        xw_ref[...] = jnp.dot(xb, wb, preferred_element_type=jnp.float32
                              ).astype(jnp.bfloat16)
        w3r = w3_ref[...]
        vab = jax.lax.dot_general(
            w3r[:, :h], w2_ref[...], (((1,), (1,)), ((), ())),
            preferred_element_type=jnp.float32)
        zrow = jnp.zeros((6, 128), jnp.float32)
        fold_ref[...] = jnp.concatenate(
            [vab[:, :128], vab[:, 128:256], zrow,
             vab[:, 256:384], vab[:, 384:], zrow,
             w3r[:, h:h + 128], w3r[:, h + 128:], zrow], axis=0)

    @pl.when(t < n_enc)
    def _():
        ab = adj_ref[...].astype(jnp.bfloat16)
        zb = jnp.dot(ab, xw_ref[...],
                     preferred_element_type=jnp.float32).astype(jnp.bfloat16)
        zb3 = zb.astype(jnp.float32).reshape(tm, 2, 128)
        z3_ref[pl.ds(t * tm, tm)] = jnp.pad(zb3, ((0, 0), (0, 6), (0, 0)))

    @pl.when(t >= n_enc)
    def _():
        for e in range(te):
            ti_ref[e] = z3_ref[ei_ref[0, 0, e]]
            tj_ref[e] = z3_ref[ej_ref[0, 0, e]]
        zi = ti_ref[...]
        zj = tj_ref[...]
        v2a = fold_ref[0:8]
        v2b = fold_ref[8:16]
        w3b = fold_ref[16:24]
        m = (zi * zj * w3b
             + jnp.maximum(zi, 0.0) * v2a
             + jnp.maximum(zj, 0.0) * v2b)
        logits = jnp.sum(jnp.sum(m, axis=1), axis=1, keepdims=True)
        o_ref[...] = 1.0 / (1.0 + jnp.exp(-logits))


def _pick_tile(n, desired):
    for t in (desired, 512, 256, 128):
        if t <= n and n % t == 0 and t % 128 == 0:
            return t
    return n


def kernel(x, adj, weight, weight_two, weight_three, train_edges, train_false_edges):
    f32 = jnp.float32
    N = adj.shape[0]
    Din, H = weight.shape
    w2 = jnp.asarray(weight_two, f32)
    w3r = jnp.asarray(weight_three, f32).reshape(1, 2 * H)
    edges = jnp.concatenate([jnp.asarray(train_edges, jnp.int32),
                             jnp.asarray(train_false_edges, jnp.int32)], axis=0)
    E = edges.shape[0]
    TE = 1024
    n_tiles = int(pl.cdiv(E, TE))
    E_pad = n_tiles * TE
    edges = jnp.pad(edges, ((0, E_pad - E), (0, 0)))
    ei = edges[:, 0].reshape(n_tiles, 1, TE)
    ej = edges[:, 1].reshape(n_tiles, 1, TE)

    tm = _pick_tile(N, 512)
    n_enc = N // tm
    last_enc = n_enc - 1
    last_o = n_tiles - 1

    body = functools.partial(_fused_kernel, tm=tm, n_enc=n_enc)
    out = pl.pallas_call(
        body,
        out_shape=jax.ShapeDtypeStruct((E_pad, 1), f32),
        grid=(n_enc + n_tiles,),
        in_specs=[
            pl.BlockSpec((tm, N), lambda t: (jnp.minimum(t, last_enc), 0)),
            pl.BlockSpec((N, Din), lambda t: (0, 0)),
            pl.BlockSpec((Din, H), lambda t: (0, 0)),
            pl.BlockSpec((2 * H, H), lambda t: (0, 0)),
            pl.BlockSpec((1, 2 * H), lambda t: (0, 0)),
            pl.BlockSpec((1, 1, TE),
                         lambda t: (jnp.clip(t - n_enc, 0, last_o), 0, 0),
                         memory_space=pltpu.SMEM),
            pl.BlockSpec((1, 1, TE),
                         lambda t: (jnp.clip(t - n_enc, 0, last_o), 0, 0),
                         memory_space=pltpu.SMEM),
        ],
        out_specs=pl.BlockSpec((TE, 1),
                               lambda t: (jnp.clip(t - n_enc, 0, last_o), 0)),
        scratch_shapes=[pltpu.VMEM((N, H), jnp.bfloat16),
                        pltpu.VMEM((N, 8, 128), f32),
                        pltpu.VMEM((24, 128), f32),
                        pltpu.VMEM((TE, 8, 128), f32),
                        pltpu.VMEM((TE, 8, 128), f32)],
        compiler_params=pltpu.CompilerParams(
            dimension_semantics=("arbitrary",),
            vmem_limit_bytes=57 * 1024 * 1024),
    )(adj, x, weight, w2, w3r, ei, ej)
    return out[:E]


# speedup vs baseline: 1.7556x; 1.0120x over previous
import functools

import jax
import jax.numpy as jnp
from jax.experimental import pallas as pl
from jax.experimental.pallas import tpu as pltpu


def _fused_kernel(adj_ref, x_ref, w1_ref, w2_ref, w3_ref, ei_ref, ej_ref,
                  o_ref, xw_ref, z3_ref, fold_ref, ti_ref, tj_ref,
                  *, tm, n_enc):
    h = w1_ref.shape[1]
    te = ti_ref.shape[0]
    t = pl.program_id(0)

    @pl.when(t == 0)
    def _():
        xb = x_ref[...].astype(jnp.bfloat16)
        wb = w1_ref[...].astype(jnp.bfloat16)
        xw_ref[...] = jnp.dot(xb, wb, preferred_element_type=jnp.float32
                              ).astype(jnp.bfloat16)
        w3r = w3_ref[...]
        vab = jax.lax.dot_general(
            w3r[:, :h], w2_ref[...], (((1,), (1,)), ((), ())),
            preferred_element_type=jnp.float32)
        zrow = jnp.zeros((6, 128), jnp.float32)
        fold_ref[...] = jnp.concatenate(
            [vab[:, :128], vab[:, 128:256], zrow,
             vab[:, 256:384], vab[:, 384:], zrow,
             w3r[:, h:h + 128], w3r[:, h + 128:], zrow], axis=0)

    @pl.when(t < n_enc)
    def _():
        ab = adj_ref[...].astype(jnp.bfloat16)
        zb = jnp.dot(ab, xw_ref[...],
                     preferred_element_type=jnp.float32).astype(jnp.bfloat16)
        zb3 = zb.astype(jnp.float32).reshape(tm, 2, 128)
        z3_ref[pl.ds(t * tm, tm)] = jnp.pad(zb3, ((0, 0), (0, 6), (0, 0)))

    @pl.when(t >= n_enc)
    def _():
        for e in range(te):
            ti_ref[e] = z3_ref[ei_ref[0, 0, e]]
            tj_ref[e] = z3_ref[ej_ref[0, 0, e]]
        zi = ti_ref[...]
        zj = tj_ref[...]
        v2a = fold_ref[0:8]
        v2b = fold_ref[8:16]
        w3b = fold_ref[16:24]
        m = (zi * zj * w3b
             + jnp.maximum(zi, 0.0) * v2a
             + jnp.maximum(zj, 0.0) * v2b)
        logits = jnp.sum(jnp.sum(m, axis=1), axis=1, keepdims=True)
        o_ref[...] = 1.0 / (1.0 + jnp.exp(-logits))


def _pick_tile(n, desired):
    for t in (desired, 512, 256, 128):
        if t <= n and n % t == 0 and t % 128 == 0:
            return t
    return n


def kernel(x, adj, weight, weight_two, weight_three, train_edges, train_false_edges):
    f32 = jnp.float32
    N = adj.shape[0]
    Din, H = weight.shape
    w2 = jnp.asarray(weight_two, f32)
    w3r = jnp.asarray(weight_three, f32).reshape(1, 2 * H)
    edges = jnp.concatenate([jnp.asarray(train_edges, jnp.int32),
                             jnp.asarray(train_false_edges, jnp.int32)], axis=0)
    E = edges.shape[0]
    TE = 2048
    n_tiles = int(pl.cdiv(E, TE))
    E_pad = n_tiles * TE
    edges = jnp.pad(edges, ((0, E_pad - E), (0, 0)))
    ei = edges[:, 0].reshape(n_tiles, 1, TE)
    ej = edges[:, 1].reshape(n_tiles, 1, TE)

    tm = _pick_tile(N, 512)
    n_enc = N // tm
    last_enc = n_enc - 1
    last_o = n_tiles - 1

    body = functools.partial(_fused_kernel, tm=tm, n_enc=n_enc)
    out = pl.pallas_call(
        body,
        out_shape=jax.ShapeDtypeStruct((E_pad, 1), f32),
        grid=(n_enc + n_tiles,),
        in_specs=[
            pl.BlockSpec((tm, N), lambda t: (jnp.minimum(t, last_enc), 0)),
            pl.BlockSpec((N, Din), lambda t: (0, 0)),
            pl.BlockSpec((Din, H), lambda t: (0, 0)),
            pl.BlockSpec((2 * H, H), lambda t: (0, 0)),
            pl.BlockSpec((1, 2 * H), lambda t: (0, 0)),
            pl.BlockSpec((1, 1, TE),
                         lambda t: (jnp.clip(t - n_enc, 0, last_o), 0, 0),
                         memory_space=pltpu.SMEM),
            pl.BlockSpec((1, 1, TE),
                         lambda t: (jnp.clip(t - n_enc, 0, last_o), 0, 0),
                         memory_space=pltpu.SMEM),
        ],
        out_specs=pl.BlockSpec((TE, 1),
                               lambda t: (jnp.clip(t - n_enc, 0, last_o), 0)),
        scratch_shapes=[pltpu.VMEM((N, H), jnp.bfloat16),
                        pltpu.VMEM((N, 8, 128), f32),
                        pltpu.VMEM((24, 128), f32),
                        pltpu.VMEM((TE, 8, 128), f32),
                        pltpu.VMEM((TE, 8, 128), f32)],
        compiler_params=pltpu.CompilerParams(
            dimension_semantics=("arbitrary",),
            vmem_limit_bytes=57 * 1024 * 1024),
    )(adj, x, weight, w2, w3r, ei, ej)
    return out[:E]
